# Initial kernel scaffold; baseline (speedup 1.0000x reference)
import functools

import jax
import jax.numpy as jnp
from jax import lax
from jax.experimental import pallas as pl
from jax.experimental.pallas import tpu as pltpu
from jax.experimental.pallas import tpu_sc as plsc

_NN = 10000
_NE = 320000
_NC, _NS, _L = 2, 16, 16
_NW = _NC * _NS
_EPT = _NE // _NW
_CK = 400
_NCHUNK = _EPT // _CK
_TGRP = 5
_GPT = _CK // (_TGRP * _L)
_XROFF = 8

_f32 = jnp.float32
_i32 = jnp.int32


def _sc_gat(dout):
    dacc = dout + 1
    accw = _NN * dacc
    wlen = 4 * dout + dout
    mesh = plsc.VectorSubcoreMesh(core_axis_name="c", subcore_axis_name="s")

    @functools.partial(
        pl.kernel,
        out_type=[
            jax.ShapeDtypeStruct((_NW, accw), _f32),
            jax.ShapeDtypeStruct((_NC, 16), _f32),
        ],
        mesh=mesh,
        compiler_params=pltpu.CompilerParams(
            needs_layout_passes=False, use_tc_tiling_on_sc=False),
        scratch_types=[
            pltpu.VMEM((_CK,), _i32),
            pltpu.VMEM((_CK,), _i32),
            pltpu.VMEM((_EPT,), _f32),
            pltpu.VMEM((_CK, 16), _f32),
            pltpu.VMEM((_CK, 16), _f32),
            pltpu.VMEM((_CK, 4), _f32),
            pltpu.VMEM((accw,), _f32),
            pltpu.VMEM((wlen,), _f32),
            pltpu.VMEM((16,), _f32),
            pltpu.VMEM((_NS, 16), _f32),
            pltpu.VMEM_SHARED((_NS, 16), _f32),
            pltpu.SemaphoreType.DMA,
            pltpu.SemaphoreType.DMA,
        ],
    )
    def sc_layer(table, srch, dsth, attrh, wvech,
                 acc_out, gmax_out,
                 src_c, dst_c, e_v, rows_s, rows_d, attr_v, acc_v, w_v,
                 tmp16, allmax_v, shmax, sem_s, sem_d):
        cid = lax.axis_index("c")
        sid = lax.axis_index("s")
        wid = sid * _NC + cid
        iota = lax.iota(_i32, _L)

        pltpu.sync_copy(wvech, w_v)

        def zbody(j, cr):
            acc_v[pl.ds(j * _L, _L)] = jnp.zeros((_L,), _f32)
            return cr
        lax.fori_loop(0, accw // _L, zbody, 0)

        def splat(k):
            return plsc.load_gather(w_v, [jnp.full((_L,), k, _i32)])
        wsp = [[splat(k * dout + c) for c in range(dout)] for k in range(4)]
        asp = [splat(4 * dout + c) for c in range(dout)]

        def cfull(v):
            return jnp.full((_L,), v, _i32)

        def pa_chunk(j, m):
            gb = wid * _EPT + j * _CK
            pltpu.sync_copy(srch.at[pl.ds(gb, _CK)], src_c)
            pltpu.sync_copy(dsth.at[pl.ds(gb, _CK)], dst_c)
            pltpu.sync_copy(attrh.at[pl.ds(gb, _CK)], attr_v)
            cs = pltpu.async_copy(table.at[src_c], rows_s, sem_s)
            cd = pltpu.async_copy(table.at[dst_c], rows_d, sem_d)
            cs.wait()
            cd.wait()

            def inner(t, mi):
                for g in range(_GPT):
                    off = t * (_GPT * _L) + g * _L
                    lidx = iota + off
                    a = [plsc.load_gather(attr_v, [lidx, cfull(k)])
                         for k in range(4)]
                    e = None
                    for c in range(dout):
                        xl = plsc.load_gather(rows_s, [lidx, cfull(c)])
                        xr = plsc.load_gather(rows_d, [lidx, cfull(_XROFF + c)])
                        ea = (a[0] * wsp[0][c] + a[1] * wsp[1][c]
                              + a[2] * wsp[2][c] + a[3] * wsp[3][c])
                        s = xl + xr + ea
                        s = jnp.maximum(s, 0.2 * s)
                        t_c = s * asp[c]
                        e = t_c if e is None else e + t_c
                    e_v[pl.ds(j * _CK + off, _L)] = e
                    mi = jnp.maximum(mi, e)
                return mi
            return lax.fori_loop(0, _TGRP, inner, m)

        m = lax.fori_loop(0, _NCHUNK, pa_chunk,
                          jnp.full((_L,), -3.0e38, _f32))

        tmp16[...] = m
        pltpu.sync_copy(tmp16, shmax.at[sid])
        plsc.subcore_barrier()
        pltpu.sync_copy(shmax, allmax_v)
        gv = allmax_v[0]
        for r in range(1, _NS):
            gv = jnp.maximum(gv, allmax_v[r])
        gmax = jnp.max(gv)

        @pl.when(sid == 0)
        def _():
            tmp16[...] = jnp.full((_L,), gmax, _f32)
            pltpu.sync_copy(tmp16, gmax_out.at[cid])

        def pb_chunk(j, cr):
            gb = wid * _EPT + j * _CK
            pltpu.sync_copy(srch.at[pl.ds(gb, _CK)], src_c)
            pltpu.sync_copy(dsth.at[pl.ds(gb, _CK)], dst_c)
            pltpu.async_copy(table.at[src_c], rows_s, sem_s).wait()

            def inner(t, cr2):
                for g in range(_GPT):
                    off = t * (_GPT * _L) + g * _L
                    lidx = iota + off
                    d16 = dst_c[pl.ds(off, _L)]
                    e16 = e_v[pl.ds(j * _CK + off, _L)]
                    w = jnp.exp(e16 - gmax)
                    ab = d16 * dacc
                    plsc.addupdate_scatter(acc_v, [ab + dout], w)
                    for c in range(dout):
                        xl = plsc.load_gather(rows_s, [lidx, cfull(c)])
                        plsc.addupdate_scatter(acc_v, [ab + c], xl * w)
                return cr2
            lax.fori_loop(0, _TGRP, inner, 0)
            return cr
        lax.fori_loop(0, _NCHUNK, pb_chunk, 0)

        pltpu.sync_copy(acc_v, acc_out.at[wid])

    return sc_layer


def _tc_call(body, out_shape, *args):
    return pl.pallas_call(
        body,
        out_shape=out_shape,
        compiler_params=pltpu.CompilerParams(
            vmem_limit_bytes=128 * 1024 * 1024),
    )(*args)


def _pre_body(x_ref, w_ref, b_ref, o_ref):
    o_ref[...] = (jnp.dot(x_ref[...], w_ref[...],
                          preferred_element_type=_f32) + b_ref[...])


def _combine(acc, gmax, bias, nw, nb, nms, dout):
    sm = jnp.sum(acc, axis=0)
    g = jnp.max(gmax)
    f = jnp.exp(gmax - g)
    f0 = f[0:1, 0:1]
    f1 = f[1:2, 0:1]
    num = sm[0, :, :dout] * f0 + sm[1, :, :dout] * f1
    den = sm[0, :, dout:] * f0 + sm[1, :, dout:] * f1
    out = num / (den + 1e-16) + bias
    mean = jnp.mean(out, axis=0, keepdims=True)
    o2 = out - nms * mean
    var = jnp.mean(o2 * o2, axis=0, keepdims=True)
    return jnp.maximum(nw * o2 / jnp.sqrt(var + 1e-5) + nb, 0.0)


def _mid_body(dout):
    def body(acc_ref, gmax_ref, bias_ref, nw_ref, nb_ref, nms_ref,
             wcat_ref, bcat_ref, h_ref, tbl_ref):
        h = _combine(acc_ref[...], gmax_ref[...], bias_ref[...],
                     nw_ref[...], nb_ref[...], nms_ref[...], dout)
        h_ref[...] = h
        tbl_ref[...] = (jnp.dot(h, wcat_ref[...],
                                preferred_element_type=_f32) + bcat_ref[...])
    return body


def _mha(q, kv, m):
    wq, bq, wk, bk, wv, bv, wo, bo = m[:8]
    qq = jnp.dot(q, wq, preferred_element_type=_f32) + bq
    kk = jnp.dot(kv, wk, preferred_element_type=_f32) + bk
    vv = jnp.dot(kv, wv, preferred_element_type=_f32) + bv
    sc = 1.0 / (3.0 ** 0.5)
    outs = []
    for h in range(6):
        qh = qq[:, 3 * h:3 * h + 3]
        kh = kk[:, 3 * h:3 * h + 3]
        vh = vv[:, 3 * h:3 * h + 3]
        lg = lax.dot_general(qh, kh, (((1,), (1,)), ((), ())),
                             preferred_element_type=_f32) * sc
        lg = lg - jnp.max(lg, axis=1, keepdims=True)
        aw = jnp.exp(lg)
        aw = aw / jnp.sum(aw, axis=1, keepdims=True)
        outs.append(jnp.dot(aw, vh, preferred_element_type=_f32))
    o = jnp.concatenate(outs, axis=1)
    return jnp.dot(o, wo, preferred_element_type=_f32) + bo


def _mabf(x, y, m):
    o = _mha(x, y, m) + x
    wl, bl = m[8], m[9]
    return o + jnp.maximum(jnp.dot(o, wl, preferred_element_type=_f32) + bl,
                           0.0)


def _final_body(*a):
    pos = 0

    def nxt(n=None):
        nonlocal pos
        if n is None:
            pos += 1
            return a[pos - 1][...]
        r = [x[...] for x in a[pos:pos + n]]
        pos += n
        return r

    acc = nxt(); gmax = nxt(); bias3 = nxt()
    n3w = nxt(); n3b = nxt(); n3ms = nxt()
    h1 = nxt(); h2 = nxt()
    advw = nxt(); advb = nxt(); adv2w = nxt(); adv2b = nxt()
    p1w = nxt(); p1b = nxt(); sd1 = nxt()
    mab1 = nxt(10)
    sab = nxt(10)
    p2w = nxt(); p2b = nxt(); sd2 = nxt()
    mab2 = nxt(10)
    valw = nxt(); valb = nxt(); val2w = nxt(); val2b = nxt()
    q_ref = a[pos]

    x3 = _combine(acc, gmax, bias3, n3w, n3b, n3ms, 4)
    xc = jnp.concatenate([h1, h2, x3], axis=1)
    adv = jnp.maximum(jnp.dot(xc, advw, preferred_element_type=_f32) + advb,
                      0.0)
    adv = jnp.dot(adv, adv2w, preferred_element_type=_f32) + adv2b
    hh = jnp.maximum(jnp.dot(xc, p1w, preferred_element_type=_f32) + p1b, 0.0)
    h1g = _mabf(sd1, hh, mab1)
    h2g = _mabf(h1g, h1g, sab)
    h2l = jnp.maximum(jnp.dot(h2g, p2w, preferred_element_type=_f32) + p2b,
                      0.0)
    pooled = _mabf(sd2, h2l, mab2)
    comb = jnp.concatenate([jnp.broadcast_to(pooled, (_NN, 18)), xc], axis=1)
    val = jnp.maximum(jnp.dot(comb, valw, preferred_element_type=_f32) + valb,
                      0.0)
    val = jnp.dot(val, val2w, preferred_element_type=_f32) + val2b
    q_ref[...] = val + adv - jnp.mean(adv)


def _wcat(pc, din, dout):
    w = jnp.zeros((din, 16), _f32)
    w = w.at[:, :dout].set(pc['Wl']).at[:, _XROFF:_XROFF + dout].set(pc['Wr'])
    b = jnp.zeros((1, 16), _f32)
    b = b.at[0, :dout].set(pc['bl']).at[0, _XROFF:_XROFF + dout].set(pc['br'])
    return w, b


def _wvec(pc):
    return jnp.concatenate([pc['We'].reshape(-1), pc['att']])


def _r2(v):
    return v.reshape(1, -1)


def kernel(x, edge_index, edge_attr, batch, params):
    p = params
    src = edge_index[0]
    dst = edge_index[1]
    sds = jax.ShapeDtypeStruct

    douts = (8, 6, 4)
    convs = (p['conv1'], p['conv2'], p['conv3'])
    norms = (p['norm1'], p['norm2'], p['norm3'])

    w1, b1 = _wcat(p['conv1'], 128, 8)
    table = _tc_call(_pre_body, sds((_NN, 16), _f32), x, w1, b1)

    hs = []
    accs = []
    gms = []
    for i in range(3):
        dout = douts[i]
        acc, gm = _sc_gat(dout)(table, src, dst, edge_attr, _wvec(convs[i]))
        acc = acc.reshape(_NS, _NC, _NN, dout + 1)
        accs.append(acc)
        gms.append(gm)
        if i < 2:
            dnext = douts[i + 1]
            wn, bn = _wcat(convs[i + 1], dout, dnext)
            h, table = _tc_call(
                _mid_body(dout),
                [sds((_NN, dout), _f32), sds((_NN, 16), _f32)],
                acc, gm, _r2(convs[i]['bias']), _r2(norms[i]['w']),
                _r2(norms[i]['b']), _r2(norms[i]['ms']), wn, bn)
            hs.append(h)

    def mabargs(m):
        return [m['Wq'], _r2(m['bq']), m['Wk'], _r2(m['bk']),
                m['Wv'], _r2(m['bv']), m['Wo'], _r2(m['bo']),
                m['Wlin'], _r2(m['blin'])]

    q2 = _tc_call(
        _final_body, sds((_NN, 1), _f32),
        accs[2], gms[2], _r2(convs[2]['bias']), _r2(norms[2]['w']),
        _r2(norms[2]['b']), _r2(norms[2]['ms']),
        hs[0], hs[1],
        p['adv_W'], _r2(p['adv_b']), p['adv2_W'], _r2(p['adv2_b']),
        p['pma1_lin_W'], _r2(p['pma1_lin_b']), p['pma1_seed'],
        *mabargs(p['pma1_mab']), *mabargs(p['sab_mab']),
        p['pma2_lin_W'], _r2(p['pma2_lin_b']), p['pma2_seed'],
        *mabargs(p['pma2_mab']),
        p['val_W'], _r2(p['val_b']), p['val2_W'], _r2(p['val2_b']))
    return q2[:, 0]


# speedup vs baseline: 19.0985x; 19.0985x over previous
import functools

import jax
import jax.numpy as jnp
from jax import lax
from jax.experimental import pallas as pl
from jax.experimental.pallas import tpu as pltpu
from jax.experimental.pallas import tpu_sc as plsc

_NN = 10000
_NE = 320000
_NC, _NS, _L = 2, 16, 16
_NW = _NC * _NS
_EPT = _NE // _NW
_CK = 400
_NCHUNK = _EPT // _CK
_TGRP = 5
_GPT = _CK // (_TGRP * _L)
_XROFF = 8

_f32 = jnp.float32
_i32 = jnp.int32


def _sc_gat(dout):
    dacc = dout + 1
    accw = _NN * dacc
    wlen = 4 * dout + dout
    mesh = plsc.VectorSubcoreMesh(core_axis_name="c", subcore_axis_name="s")

    @functools.partial(
        pl.kernel,
        out_type=[
            jax.ShapeDtypeStruct((_NW, accw), _f32),
            jax.ShapeDtypeStruct((_NC, 16), _f32),
        ],
        mesh=mesh,
        compiler_params=pltpu.CompilerParams(
            needs_layout_passes=False, use_tc_tiling_on_sc=False),
        scratch_types=[
            pltpu.VMEM((_CK,), _i32),
            pltpu.VMEM((_CK,), _i32),
            pltpu.VMEM((_EPT,), _f32),
            pltpu.VMEM((_CK, 16), _f32),
            pltpu.VMEM((_CK, 16), _f32),
            pltpu.VMEM((_CK, 4), _f32),
            pltpu.VMEM((accw,), _f32),
            pltpu.VMEM((wlen, 16), _f32),
            pltpu.VMEM((16,), _f32),
            pltpu.VMEM((_NS, 16), _f32),
            pltpu.VMEM_SHARED((_NS, 16), _f32),
            pltpu.SemaphoreType.DMA,
            pltpu.SemaphoreType.DMA,
        ],
    )
    def sc_layer(table, srch, dsth, attrh, wvech,
                 acc_out, gmax_out,
                 src_c, dst_c, e_v, rows_s, rows_d, attr_v, acc_v, w_v,
                 tmp16, allmax_v, shmax, sem_s, sem_d):
        cid = lax.axis_index("c")
        sid = lax.axis_index("s")
        wid = sid * _NC + cid
        iota = lax.iota(_i32, _L)

        pltpu.sync_copy(wvech, w_v)

        def zbody(j, cr):
            acc_v[pl.ds(j * _L, _L)] = jnp.zeros((_L,), _f32)
            return cr
        lax.fori_loop(0, accw // _L, zbody, 0)

        def bround(v):
            u = plsc.bitcast(v, _i32)
            r = (u + 32767 + ((u >> 16) & 1)) & (-65536)
            return plsc.bitcast(r, _f32)

        def splat(k):
            return bround(w_v[k])
        wsp = [[splat(k * dout + c) for c in range(dout)] for k in range(4)]
        asp = [splat(4 * dout + c) for c in range(dout)]

        def cfull(v):
            return jnp.full((_L,), v, _i32)

        def pa_chunk(j, m):
            gb = wid * _EPT + j * _CK
            pltpu.sync_copy(srch.at[pl.ds(gb, _CK)], src_c)
            pltpu.sync_copy(dsth.at[pl.ds(gb, _CK)], dst_c)
            pltpu.sync_copy(attrh.at[pl.ds(gb, _CK)], attr_v)
            cs = pltpu.async_copy(table.at[src_c], rows_s, sem_s)
            cd = pltpu.async_copy(table.at[dst_c], rows_d, sem_d)
            cs.wait()
            cd.wait()

            def inner(t, mi):
                for g in range(_GPT):
                    off = t * (_GPT * _L) + g * _L
                    lidx = iota + off
                    a = [bround(plsc.load_gather(attr_v, [lidx, cfull(k)]))
                         for k in range(4)]
                    e = None
                    for c in range(dout):
                        xl = plsc.load_gather(rows_s, [lidx, cfull(c)])
                        xr = plsc.load_gather(rows_d, [lidx, cfull(_XROFF + c)])
                        ea = (a[0] * wsp[0][c] + a[1] * wsp[1][c]
                              + a[2] * wsp[2][c] + a[3] * wsp[3][c])
                        s = xl + xr + ea
                        s = jnp.maximum(s, 0.2 * s)
                        t_c = bround(s) * asp[c]
                        e = t_c if e is None else e + t_c
                    e_v[pl.ds(j * _CK + off, _L)] = e
                    mi = jnp.maximum(mi, e)
                return mi
            return lax.fori_loop(0, _TGRP, inner, m)

        m = lax.fori_loop(0, _NCHUNK, pa_chunk,
                          jnp.full((_L,), -3.0e38, _f32))

        tmp16[...] = m
        pltpu.sync_copy(tmp16, shmax.at[sid])
        plsc.subcore_barrier()
        pltpu.sync_copy(shmax, allmax_v)
        gv = allmax_v[0]
        for r in range(1, _NS):
            gv = jnp.maximum(gv, allmax_v[r])
        gmax = jnp.max(gv)

        @pl.when(sid == 0)
        def _():
            tmp16[...] = jnp.full((_L,), gmax, _f32)
            pltpu.sync_copy(tmp16, gmax_out.at[cid])

        def pb_chunk(j, cr):
            gb = wid * _EPT + j * _CK
            pltpu.sync_copy(srch.at[pl.ds(gb, _CK)], src_c)
            pltpu.sync_copy(dsth.at[pl.ds(gb, _CK)], dst_c)
            pltpu.async_copy(table.at[src_c], rows_s, sem_s).wait()

            def inner(t, cr2):
                for g in range(_GPT):
                    off = t * (_GPT * _L) + g * _L
                    lidx = iota + off
                    d16 = dst_c[pl.ds(off, _L)]
                    e16 = e_v[pl.ds(j * _CK + off, _L)]
                    w = jnp.exp(e16 - gmax)
                    plsc.addupdate_scatter(acc_v, [d16 + dout * _NN], w)
                    for c in range(dout):
                        xl = plsc.load_gather(rows_s, [lidx, cfull(c)])
                        plsc.addupdate_scatter(acc_v, [d16 + c * _NN], xl * w)
                return cr2
            lax.fori_loop(0, _TGRP, inner, 0)
            return cr
        lax.fori_loop(0, _NCHUNK, pb_chunk, 0)

        pltpu.sync_copy(acc_v, acc_out.at[wid])

    return sc_layer


def _tc_call(body, out_shape, *args):
    return pl.pallas_call(
        body,
        out_shape=out_shape,
        compiler_params=pltpu.CompilerParams(
            vmem_limit_bytes=128 * 1024 * 1024),
    )(*args)


def _pre_body(x_ref, w_ref, b_ref, o_ref):
    o_ref[...] = (jnp.dot(x_ref[...], w_ref[...],
                          preferred_element_type=_f32) + b_ref[...])


def _combine(acc, gmax, bias, nw, nb, nms, dout):
    sm = jnp.sum(acc, axis=0)
    s0 = sm[0]
    s1 = sm[1]
    g = jnp.max(gmax)
    f = jnp.exp(gmax - g)
    f0 = f[0:1, 0:1]
    f1 = f[1:2, 0:1]
    num = s0[:dout] * f0 + s1[:dout] * f1
    den = s0[dout:] * f0 + s1[dout:] * f1
    out = num / (den + 1e-16) + bias
    mean = jnp.mean(out, axis=1, keepdims=True)
    o2 = out - nms * mean
    var = jnp.mean(o2 * o2, axis=1, keepdims=True)
    return jnp.maximum(nw * o2 / jnp.sqrt(var + 1e-5) + nb, 0.0)


def _dotc(cm, w):
    return lax.dot_general(cm, w, (((0,), (0,)), ((), ())),
                           preferred_element_type=_f32)


def _mid_body(dout):
    def body(acc_ref, gmax_ref, bias_ref, nw_ref, nb_ref, nms_ref,
             wcat_ref, bcat_ref, h_ref, tbl_ref):
        h = _combine(acc_ref[...], gmax_ref[...], bias_ref[...],
                     nw_ref[...], nb_ref[...], nms_ref[...], dout)
        h_ref[...] = h
        tbl_ref[...] = _dotc(h, wcat_ref[...]) + bcat_ref[...]
    return body


def _mha(q, kv, m):
    wq, bq, wk, bk, wv, bv, wo, bo = m[:8]
    qq = jnp.dot(q, wq, preferred_element_type=_f32) + bq
    kk = jnp.dot(kv, wk, preferred_element_type=_f32) + bk
    vv = jnp.dot(kv, wv, preferred_element_type=_f32) + bv
    sc = 1.0 / (3.0 ** 0.5)
    outs = []
    for h in range(6):
        qh = qq[:, 3 * h:3 * h + 3]
        kh = kk[:, 3 * h:3 * h + 3]
        vh = vv[:, 3 * h:3 * h + 3]
        lg = lax.dot_general(qh, kh, (((1,), (1,)), ((), ())),
                             preferred_element_type=_f32) * sc
        lg = lg - jnp.max(lg, axis=1, keepdims=True)
        aw = jnp.exp(lg)
        aw = aw / jnp.sum(aw, axis=1, keepdims=True)
        outs.append(jnp.dot(aw, vh, preferred_element_type=_f32))
    o = jnp.concatenate(outs, axis=1)
    return jnp.dot(o, wo, preferred_element_type=_f32) + bo


def _mabf(x, y, m):
    o = _mha(x, y, m) + x
    wl, bl = m[8], m[9]
    return o + jnp.maximum(jnp.dot(o, wl, preferred_element_type=_f32) + bl,
                           0.0)


def _final_body(*a):
    pos = 0

    def nxt(n=None):
        nonlocal pos
        if n is None:
            pos += 1
            return a[pos - 1][...]
        r = [x[...] for x in a[pos:pos + n]]
        pos += n
        return r

    acc = nxt(); gmax = nxt(); bias3 = nxt()
    n3w = nxt(); n3b = nxt(); n3ms = nxt()
    h1 = nxt(); h2 = nxt()
    advw = nxt(); advb = nxt(); adv2w = nxt(); adv2b = nxt()
    p1w = nxt(); p1b = nxt(); sd1 = nxt()
    mab1 = nxt(10)
    sab = nxt(10)
    p2w = nxt(); p2b = nxt(); sd2 = nxt()
    mab2 = nxt(10)
    valwp = nxt(); valwx = nxt(); valb = nxt()
    val2w = nxt(); val2b = nxt()
    q_ref = a[pos]

    x3 = _combine(acc, gmax, bias3, n3w, n3b, n3ms, 4)
    xc = jnp.concatenate([h1, h2, x3], axis=0)
    adv = jnp.maximum(_dotc(xc, advw) + advb, 0.0)
    adv = jnp.dot(adv, adv2w, preferred_element_type=_f32) + adv2b
    hh = jnp.maximum(_dotc(xc, p1w) + p1b, 0.0)
    h1g = _mabf(sd1, hh, mab1)
    h2g = _mabf(h1g, h1g, sab)
    h2l = jnp.maximum(jnp.dot(h2g, p2w, preferred_element_type=_f32) + p2b,
                      0.0)
    pooled = _mabf(sd2, h2l, mab2)
    val = jnp.maximum(
        jnp.dot(pooled, valwp, preferred_element_type=_f32)
        + _dotc(xc, valwx) + valb, 0.0)
    val = jnp.dot(val, val2w, preferred_element_type=_f32) + val2b
    q_ref[...] = val + adv - jnp.mean(adv)


def _wcat(pc, din, dout):
    w = jnp.zeros((din, 16), _f32)
    w = w.at[:, :dout].set(pc['Wl']).at[:, _XROFF:_XROFF + dout].set(pc['Wr'])
    b = jnp.zeros((1, 16), _f32)
    b = b.at[0, :dout].set(pc['bl']).at[0, _XROFF:_XROFF + dout].set(pc['br'])
    return w, b


def _wvec(pc):
    v = jnp.concatenate([pc['We'].reshape(-1), pc['att']])
    return jnp.tile(v[:, None], (1, 16))


def _r2(v):
    return v.reshape(1, -1)


def _c2(v):
    return v.reshape(-1, 1)


def kernel(x, edge_index, edge_attr, batch, params):
    p = params
    src = edge_index[0]
    dst = edge_index[1]
    sds = jax.ShapeDtypeStruct

    douts = (8, 6, 4)
    convs = (p['conv1'], p['conv2'], p['conv3'])
    norms = (p['norm1'], p['norm2'], p['norm3'])

    w1, b1 = _wcat(p['conv1'], 128, 8)
    table = _tc_call(_pre_body, sds((_NN, 16), _f32), x, w1, b1)

    hs = []
    accs = []
    gms = []
    for i in range(3):
        dout = douts[i]
        acc, gm = _sc_gat(dout)(table, src, dst, edge_attr, _wvec(convs[i]))
        acc = acc.reshape(_NS, _NC, dout + 1, _NN)
        accs.append(acc)
        gms.append(gm)
        if i < 2:
            dnext = douts[i + 1]
            wn, bn = _wcat(convs[i + 1], dout, dnext)
            h, table = _tc_call(
                _mid_body(dout),
                [sds((dout, _NN), _f32), sds((_NN, 16), _f32)],
                acc, gm, _c2(convs[i]['bias']), _c2(norms[i]['w']),
                _c2(norms[i]['b']), _c2(norms[i]['ms']), wn, bn)
            hs.append(h)

    def mabargs(m):
        return [m['Wq'], _r2(m['bq']), m['Wk'], _r2(m['bk']),
                m['Wv'], _r2(m['bv']), m['Wo'], _r2(m['bo']),
                m['Wlin'], _r2(m['blin'])]

    q2 = _tc_call(
        _final_body, sds((_NN, 1), _f32),
        accs[2], gms[2], _c2(convs[2]['bias']), _c2(norms[2]['w']),
        _c2(norms[2]['b']), _c2(norms[2]['ms']),
        hs[0], hs[1],
        p['adv_W'], _r2(p['adv_b']), p['adv2_W'], _r2(p['adv2_b']),
        p['pma1_lin_W'], _r2(p['pma1_lin_b']), p['pma1_seed'],
        *mabargs(p['pma1_mab']), *mabargs(p['sab_mab']),
        p['pma2_lin_W'], _r2(p['pma2_lin_b']), p['pma2_seed'],
        *mabargs(p['pma2_mab']),
        p['val_W'][:18], p['val_W'][18:], _r2(p['val_b']),
        p['val2_W'], _r2(p['val2_b']))
    return q2[:, 0]
